# baseline (speedup 1.0000x reference)
import functools

import jax
import jax.numpy as jnp
from jax import lax
from jax.experimental import pallas as pl
from jax.experimental.pallas import tpu as pltpu
from jax.experimental.pallas import tpu_sc as plsc

N = 10000
E = 160000
D = 256
HALF = 128
PW = 64
G = 64
C = 10

NP = 10240
EP = 163840
BR = 512
NTILES = 16
RPT = NP // NTILES
NCH_H = 40
CH = 64
NCH_A = 160
NBUF = 4

f32 = jnp.float32
u32 = jnp.uint32
i32 = jnp.int32


def _hist_body(dst_hbm, ones_hbm, zeros_hbm, out_hbm, idx_v, ones_v, hist_sp):
    c = lax.axis_index("c")
    s = lax.axis_index("s")
    wid = c * NTILES + s
    pltpu.sync_copy(ones_hbm, ones_v)
    pltpu.sync_copy(zeros_hbm.at[pl.ds(s * RPT, RPT)],
                    hist_sp.at[pl.ds(s * RPT, RPT)])
    pltpu.sync_copy(dst_hbm.at[wid], idx_v)
    plsc.subcore_barrier()

    def body(j, carry):
        pltpu.sync_copy(ones_v, hist_sp.at[idx_v.at[j]], add=True)
        return carry

    lax.fori_loop(0, NCH_H, body, 0)
    plsc.subcore_barrier()
    pltpu.sync_copy(hist_sp.at[pl.ds(s * RPT, RPT)],
                    out_hbm.at[c, pl.ds(s * RPT, RPT)])


def _agg_body(ya_hbm, yb_hbm, src_hbm, dst_hbm, zeros_hbm, outa, outb,
              srcv, dstv, rows, stag, s_sp, semg0, semg1, semg2, semg3):
    c = lax.axis_index("c")
    s = lax.axis_index("s")
    pltpu.sync_copy(zeros_hbm.at[pl.ds(s * RPT, RPT)],
                    s_sp.at[pl.ds(s * RPT, RPT)])
    plsc.subcore_barrier()
    semg = (semg0, semg1, semg2, semg3)
    stage_ch = NCH_A // 2

    def unpack_chunk(p):
        UR = 4

        def rowbody(r4, carry):
            mask_hi = jnp.full((16,), -65536, i32)
            sixteen = jnp.full((16,), 16, i32)
            r = r4 * UR
            for dr in range(UR):
                for g in range(PW // 16):
                    w = rows[p * CH + r + dr, pl.ds(g * 16, 16)]
                    lo = lax.bitcast_convert_type(
                        lax.shift_left(w, sixteen), f32)
                    hi = lax.bitcast_convert_type(
                        jnp.bitwise_and(w, mask_hi), f32)
                    stag[r + dr, pl.ds(g * 16, 16)] = lo
                    stag[r + dr, pl.ds(PW + g * 16, 16)] = hi
            return carry

        lax.fori_loop(0, CH // UR, rowbody, 0)

    def run(y_ref, out_ref):
        for h in range(2):
            pltpu.sync_copy(src_hbm.at[s, pl.ds(h * stage_ch, stage_ch)], srcv)
            pltpu.sync_copy(dst_hbm.at[s, pl.ds(h * stage_ch, stage_ch)], dstv)
            for p in range(NBUF - 1):
                pltpu.async_copy(y_ref.at[srcv.at[p]], rows.at[pl.ds(p * CH, CH)], semg[p])

            def outer(k, carry):
                for p in range(NBUF):
                    j = NBUF * k + p
                    pltpu.make_async_copy(y_ref.at[srcv.at[j]],
                                          rows.at[pl.ds(p * CH, CH)],
                                          semg[p]).wait()
                    nxt = j + NBUF - 1

                    @pl.when(nxt < stage_ch)
                    def _():
                        q = (p + NBUF - 1) % NBUF
                        pltpu.async_copy(y_ref.at[srcv.at[nxt]],
                                         rows.at[pl.ds(q * CH, CH)], semg[q])

                    unpack_chunk(p)
                    pltpu.sync_copy(stag, s_sp.at[dstv.at[j]], add=True)
                return carry

            lax.fori_loop(0, stage_ch // NBUF, outer, 0)
        plsc.subcore_barrier()
        pltpu.sync_copy(s_sp.at[pl.ds(s * RPT, RPT)],
                        out_ref.at[pl.ds(s * RPT, RPT)])

    @pl.when(c == 0)
    def _():
        run(ya_hbm, outa)

    @pl.when(c == 1)
    def _():
        run(yb_hbm, outb)


@functools.cache
def _sc_kernels():
    mesh = plsc.VectorSubcoreMesh(core_axis_name="c", subcore_axis_name="s")
    hist = pl.kernel(
        _hist_body,
        out_type=jax.ShapeDtypeStruct((2, NP, HALF), f32),
        mesh=mesh,
        scratch_types=[
            pltpu.VMEM((NCH_H, 128), i32),
            pltpu.VMEM((128, HALF), f32),
            pltpu.VMEM_SHARED((NP, HALF), f32),
        ],
    )
    agg = pl.kernel(
        _agg_body,
        out_type=[jax.ShapeDtypeStruct((NP, HALF), f32),
                  jax.ShapeDtypeStruct((NP, HALF), f32)],
        mesh=mesh,
        compiler_params=pltpu.CompilerParams(use_tc_tiling_on_sc=False),
        scratch_types=[
            pltpu.VMEM((NCH_A // 2, CH), i32),
            pltpu.VMEM((NCH_A // 2, CH), i32),
            pltpu.VMEM((NBUF * CH, PW), i32),
            pltpu.VMEM((CH, HALF), f32),
            pltpu.VMEM_SHARED((NP, HALF), f32),
            pltpu.SemaphoreType.DMA,
            pltpu.SemaphoreType.DMA,
            pltpu.SemaphoreType.DMA,
            pltpu.SemaphoreType.DMA,
        ],
    )
    return hist, agg


def _pack_half(y_half):

    def rne(v):
        u = lax.bitcast_convert_type(v, u32)
        return (u + 0x7FFF + ((u >> 16) & 1)) >> 16

    packed = rne(y_half[:, :PW]) | (rne(y_half[:, PW:]) << 16)
    return lax.bitcast_convert_type(packed, i32)


def _unpack_half(p_ref):
    u = lax.bitcast_convert_type(p_ref[...], u32)
    lo = lax.bitcast_convert_type(u << 16, f32)
    hi = lax.bitcast_convert_type(u & u32(0xFFFF0000), f32)
    return jnp.concatenate([lo, hi], axis=1)


def _mm1_body(x_ref, w_ref, h_ref, ya_ref, yb_ref, dis_ref):
    deg = h_ref[0][:, 0:1] + h_ref[1][:, 0:1] + 1.0
    dis = lax.rsqrt(deg)
    y = jnp.dot(x_ref[...], w_ref[...], preferred_element_type=f32) * dis
    ya_ref[...] = _pack_half(y[:, :HALF])
    yb_ref[...] = _pack_half(y[:, HALF:])
    dis_ref[...] = jnp.broadcast_to(dis, (BR, HALF))


_mm1 = pl.pallas_call(
    _mm1_body,
    grid=(NP // BR,),
    in_specs=[
        pl.BlockSpec((BR, D), lambda i: (i, 0)),
        pl.BlockSpec((D, D), lambda i: (0, 0)),
        pl.BlockSpec((2, BR, HALF), lambda i: (0, i, 0)),
    ],
    out_specs=[
        pl.BlockSpec((BR, PW), lambda i: (i, 0)),
        pl.BlockSpec((BR, PW), lambda i: (i, 0)),
        pl.BlockSpec((BR, HALF), lambda i: (i, 0)),
    ],
    out_shape=[
        jax.ShapeDtypeStruct((NP, PW), i32),
        jax.ShapeDtypeStruct((NP, PW), i32),
        jax.ShapeDtypeStruct((NP, HALF), f32),
    ],
)


def _mm2_body(sa_ref, sb_ref, ya_ref, yb_ref, dis_ref, b_ref, w_ref,
              oa_ref, ob_ref):
    dis = dis_ref[...]
    ha = (sa_ref[...] + _unpack_half(ya_ref)) * dis
    hb = (sb_ref[...] + _unpack_half(yb_ref)) * dis
    h = jnp.maximum(jnp.concatenate([ha, hb], axis=1) + b_ref[...], 0.0)
    y2 = jnp.dot(h, w_ref[...], preferred_element_type=f32)
    oa_ref[...] = _pack_half(y2[:, :HALF] * dis)
    ob_ref[...] = _pack_half(y2[:, HALF:] * dis)


_mm2 = pl.pallas_call(
    _mm2_body,
    grid=(NP // BR,),
    in_specs=[
        pl.BlockSpec((BR, HALF), lambda i: (i, 0)),
        pl.BlockSpec((BR, HALF), lambda i: (i, 0)),
        pl.BlockSpec((BR, PW), lambda i: (i, 0)),
        pl.BlockSpec((BR, PW), lambda i: (i, 0)),
        pl.BlockSpec((BR, HALF), lambda i: (i, 0)),
        pl.BlockSpec((1, D), lambda i: (0, 0)),
        pl.BlockSpec((D, D), lambda i: (0, 0)),
    ],
    out_specs=[
        pl.BlockSpec((BR, PW), lambda i: (i, 0)),
        pl.BlockSpec((BR, PW), lambda i: (i, 0)),
    ],
    out_shape=[
        jax.ShapeDtypeStruct((NP, PW), i32),
        jax.ShapeDtypeStruct((NP, PW), i32),
    ],
)


def _fin_body(sa_ref, sb_ref, ya_ref, yb_ref, dis_ref, b_ref, batch_ref,
              wfc_ref, bfc_ref, out_ref, acc, cnt):
    i = pl.program_id(0)

    @pl.when(i == 0)
    def _():
        acc[...] = jnp.zeros_like(acc)
        cnt[...] = jnp.zeros_like(cnt)

    dis = dis_ref[...]
    ha = (sa_ref[...] + _unpack_half(ya_ref)) * dis
    hb = (sb_ref[...] + _unpack_half(yb_ref)) * dis
    h = jnp.maximum(jnp.concatenate([ha, hb], axis=1) + b_ref[...], 0.0)
    gid = lax.broadcasted_iota(i32, (G, BR), 0)
    onehot = (gid == batch_ref[...]).astype(f32)
    acc[...] += jnp.dot(onehot, h, preferred_element_type=f32)
    cnt[...] += jnp.broadcast_to(
        jnp.sum(onehot, axis=1, keepdims=True), (G, D))

    @pl.when(i == NP // BR - 1)
    def _():
        g = acc[...] / jnp.maximum(cnt[...], 1.0)
        out_ref[...] = (jnp.dot(g, wfc_ref[...], preferred_element_type=f32)
                        + bfc_ref[...])


_fin = pl.pallas_call(
    _fin_body,
    grid=(NP // BR,),
    in_specs=[
        pl.BlockSpec((BR, HALF), lambda i: (i, 0)),
        pl.BlockSpec((BR, HALF), lambda i: (i, 0)),
        pl.BlockSpec((BR, PW), lambda i: (i, 0)),
        pl.BlockSpec((BR, PW), lambda i: (i, 0)),
        pl.BlockSpec((BR, HALF), lambda i: (i, 0)),
        pl.BlockSpec((1, D), lambda i: (0, 0)),
        pl.BlockSpec((1, BR), lambda i: (0, i)),
        pl.BlockSpec((D, HALF), lambda i: (0, 0)),
        pl.BlockSpec((1, HALF), lambda i: (0, 0)),
    ],
    out_specs=pl.BlockSpec((G, HALF), lambda i: (0, 0)),
    out_shape=jax.ShapeDtypeStruct((G, HALF), f32),
    scratch_shapes=[pltpu.VMEM((G, D), f32), pltpu.VMEM((G, D), f32)],
)


@jax.jit
def kernel(x, edge_index, batch, W1, b1, W2, b2, Wfc, bfc):
    xp = jnp.zeros((NP, D), f32).at[:N, :].set(x.astype(f32))
    src = edge_index[0].astype(i32)
    dst = edge_index[1].astype(i32)
    srcp = jnp.full((EP,), NP - 1, i32).at[:E].set(src)
    dstp = jnp.full((EP,), NP - 1, i32).at[:E].set(dst)
    dst32 = dstp.reshape(32, NCH_H, 128)
    src16 = srcp.reshape(NTILES, NCH_A, CH)
    dst16 = dstp.reshape(NTILES, NCH_A, CH)
    ones_h = jnp.ones((128, HALF), f32)
    zeros_h = jnp.zeros((NP, HALF), f32)
    batch_row = jnp.full((1, NP), G, i32).at[0, :N].set(batch.astype(i32))
    b1r = b1.reshape(1, D).astype(f32)
    b2r = b2.reshape(1, D).astype(f32)
    wfc_p = jnp.zeros((D, HALF), f32).at[:, :C].set(Wfc.astype(f32))
    bfc_p = jnp.zeros((1, HALF), f32).at[0, :C].set(bfc.astype(f32))

    hist_k, agg_k = _sc_kernels()
    hist = hist_k(dst32, ones_h, zeros_h)
    ya, yb, dis = _mm1(xp, W1.astype(f32), hist)
    s1a, s1b = agg_k(ya, yb, src16, dst16, zeros_h)
    y2a, y2b = _mm2(s1a, s1b, ya, yb, dis, b1r, W2.astype(f32))
    s2a, s2b = agg_k(y2a, y2b, src16, dst16, zeros_h)
    outp = _fin(s2a, s2b, y2a, y2b, dis, b2r, batch_row, wfc_p, bfc_p)
    return outp[:, :C]


# speedup vs baseline: 1.0608x; 1.0608x over previous
import functools

import jax
import jax.numpy as jnp
from jax import lax
from jax.experimental import pallas as pl
from jax.experimental.pallas import tpu as pltpu
from jax.experimental.pallas import tpu_sc as plsc

N = 10000
E = 160000
D = 256
HALF = 128
PW = 64
G = 64
C = 10

NP = 10240
EP = 163840
BR = 512
NTILES = 16
RPT = NP // NTILES
NCH_H = 40
CH = 64
NCH_A = 160
NBUF = 4

f32 = jnp.float32
u32 = jnp.uint32
i32 = jnp.int32


def _hist_body(dst_hbm, ones_hbm, zeros_hbm, out_hbm, idx_v, ones_v, hist_sp):
    c = lax.axis_index("c")
    s = lax.axis_index("s")
    wid = c * NTILES + s
    pltpu.sync_copy(ones_hbm, ones_v)
    pltpu.sync_copy(zeros_hbm.at[pl.ds(s * RPT, RPT)],
                    hist_sp.at[pl.ds(s * RPT, RPT)])
    pltpu.sync_copy(dst_hbm.at[wid], idx_v)
    plsc.subcore_barrier()

    def body(j, carry):
        pltpu.sync_copy(ones_v, hist_sp.at[idx_v.at[j]], add=True)
        return carry

    lax.fori_loop(0, NCH_H, body, 0)
    plsc.subcore_barrier()
    pltpu.sync_copy(hist_sp.at[pl.ds(s * RPT, RPT)],
                    out_hbm.at[c, pl.ds(s * RPT, RPT)])


def _agg_body(ya_hbm, yb_hbm, src_hbm, dst_hbm, zeros_hbm, outa, outb,
              srcv, dstv, rows, stag, s_sp, semg0, semg1, semg2, semg3):
    c = lax.axis_index("c")
    s = lax.axis_index("s")
    pltpu.sync_copy(zeros_hbm.at[pl.ds(s * RPT, RPT)],
                    s_sp.at[pl.ds(s * RPT, RPT)])
    plsc.subcore_barrier()
    semg = (semg0, semg1, semg2, semg3)
    stage_ch = NCH_A // 2

    def unpack_chunk(p):
        def rowbody(r, carry):
            mask_hi = jnp.full((16,), -65536, i32)
            sixteen = jnp.full((16,), 16, i32)
            for g in range(PW // 16):
                w = rows[p * CH + r, pl.ds(g * 16, 16)]
                lo = lax.bitcast_convert_type(lax.shift_left(w, sixteen), f32)
                hi = lax.bitcast_convert_type(jnp.bitwise_and(w, mask_hi), f32)
                stag[r, pl.ds(g * 16, 16)] = lo
                stag[r, pl.ds(PW + g * 16, 16)] = hi
            return carry

        lax.fori_loop(0, CH, rowbody, 0)

    def run(y_ref, out_ref):
        for h in range(2):
            pltpu.sync_copy(src_hbm.at[s, pl.ds(h * stage_ch, stage_ch)], srcv)
            pltpu.sync_copy(dst_hbm.at[s, pl.ds(h * stage_ch, stage_ch)], dstv)
            for p in range(NBUF - 1):
                pltpu.async_copy(y_ref.at[srcv.at[p]], rows.at[pl.ds(p * CH, CH)], semg[p])

            def outer(k, carry):
                for p in range(NBUF):
                    j = NBUF * k + p
                    pltpu.make_async_copy(y_ref.at[srcv.at[j]],
                                          rows.at[pl.ds(p * CH, CH)],
                                          semg[p]).wait()
                    nxt = j + NBUF - 1

                    @pl.when(nxt < stage_ch)
                    def _():
                        q = (p + NBUF - 1) % NBUF
                        pltpu.async_copy(y_ref.at[srcv.at[nxt]],
                                         rows.at[pl.ds(q * CH, CH)], semg[q])

                    unpack_chunk(p)
                    pltpu.sync_copy(stag, s_sp.at[dstv.at[j]], add=True)
                return carry

            lax.fori_loop(0, stage_ch // NBUF, outer, 0)
        plsc.subcore_barrier()
        pltpu.sync_copy(s_sp.at[pl.ds(s * RPT, RPT)],
                        out_ref.at[pl.ds(s * RPT, RPT)])

    @pl.when(c == 0)
    def _():
        run(ya_hbm, outa)

    @pl.when(c == 1)
    def _():
        run(yb_hbm, outb)


@functools.cache
def _sc_kernels():
    mesh = plsc.VectorSubcoreMesh(core_axis_name="c", subcore_axis_name="s")
    hist = pl.kernel(
        _hist_body,
        out_type=jax.ShapeDtypeStruct((2, NP, 16), f32),
        mesh=mesh,
        compiler_params=pltpu.CompilerParams(use_tc_tiling_on_sc=False),
        scratch_types=[
            pltpu.VMEM((NCH_H, 128), i32),
            pltpu.VMEM((128, 16), f32),
            pltpu.VMEM_SHARED((NP, 16), f32),
        ],
    )
    agg = pl.kernel(
        _agg_body,
        out_type=[jax.ShapeDtypeStruct((NP, HALF), f32),
                  jax.ShapeDtypeStruct((NP, HALF), f32)],
        mesh=mesh,
        compiler_params=pltpu.CompilerParams(use_tc_tiling_on_sc=False),
        scratch_types=[
            pltpu.VMEM((NCH_A // 2, CH), i32),
            pltpu.VMEM((NCH_A // 2, CH), i32),
            pltpu.VMEM((NBUF * CH, PW), i32),
            pltpu.VMEM((CH, HALF), f32),
            pltpu.VMEM_SHARED((NP, HALF), f32),
            pltpu.SemaphoreType.DMA,
            pltpu.SemaphoreType.DMA,
            pltpu.SemaphoreType.DMA,
            pltpu.SemaphoreType.DMA,
        ],
    )
    return hist, agg


def _pack_half(y_half):

    def rne(v):
        u = lax.bitcast_convert_type(v, u32)
        return (u + 0x7FFF + ((u >> 16) & 1)) >> 16

    packed = rne(y_half[:, :PW]) | (rne(y_half[:, PW:]) << 16)
    return lax.bitcast_convert_type(packed, i32)


def _unpack_half(p_ref):
    u = lax.bitcast_convert_type(p_ref[...], u32)
    lo = lax.bitcast_convert_type(u << 16, f32)
    hi = lax.bitcast_convert_type(u & u32(0xFFFF0000), f32)
    return jnp.concatenate([lo, hi], axis=1)


def _mm1_body(x_ref, w_ref, h_ref, ya_ref, yb_ref, dis_ref):
    deg = h_ref[0][:, 0:1] + h_ref[1][:, 0:1] + 1.0
    dis = lax.rsqrt(deg)
    y = jnp.dot(x_ref[...], w_ref[...], preferred_element_type=f32) * dis
    ya_ref[...] = _pack_half(y[:, :HALF])
    yb_ref[...] = _pack_half(y[:, HALF:])
    dis_ref[...] = jnp.broadcast_to(dis, (BR, HALF))


_mm1 = pl.pallas_call(
    _mm1_body,
    grid=(NP // BR,),
    in_specs=[
        pl.BlockSpec((BR, D), lambda i: (i, 0)),
        pl.BlockSpec((D, D), lambda i: (0, 0)),
        pl.BlockSpec((2, BR, 16), lambda i: (0, i, 0)),
    ],
    out_specs=[
        pl.BlockSpec((BR, PW), lambda i: (i, 0)),
        pl.BlockSpec((BR, PW), lambda i: (i, 0)),
        pl.BlockSpec((BR, HALF), lambda i: (i, 0)),
    ],
    out_shape=[
        jax.ShapeDtypeStruct((NP, PW), i32),
        jax.ShapeDtypeStruct((NP, PW), i32),
        jax.ShapeDtypeStruct((NP, HALF), f32),
    ],
)


def _mm2_body(sa_ref, sb_ref, ya_ref, yb_ref, dis_ref, b_ref, w_ref,
              oa_ref, ob_ref):
    dis = dis_ref[...]
    ha = (sa_ref[...] + _unpack_half(ya_ref)) * dis
    hb = (sb_ref[...] + _unpack_half(yb_ref)) * dis
    h = jnp.maximum(jnp.concatenate([ha, hb], axis=1) + b_ref[...], 0.0)
    y2 = jnp.dot(h, w_ref[...], preferred_element_type=f32)
    oa_ref[...] = _pack_half(y2[:, :HALF] * dis)
    ob_ref[...] = _pack_half(y2[:, HALF:] * dis)


_mm2 = pl.pallas_call(
    _mm2_body,
    grid=(NP // BR,),
    in_specs=[
        pl.BlockSpec((BR, HALF), lambda i: (i, 0)),
        pl.BlockSpec((BR, HALF), lambda i: (i, 0)),
        pl.BlockSpec((BR, PW), lambda i: (i, 0)),
        pl.BlockSpec((BR, PW), lambda i: (i, 0)),
        pl.BlockSpec((BR, HALF), lambda i: (i, 0)),
        pl.BlockSpec((1, D), lambda i: (0, 0)),
        pl.BlockSpec((D, D), lambda i: (0, 0)),
    ],
    out_specs=[
        pl.BlockSpec((BR, PW), lambda i: (i, 0)),
        pl.BlockSpec((BR, PW), lambda i: (i, 0)),
    ],
    out_shape=[
        jax.ShapeDtypeStruct((NP, PW), i32),
        jax.ShapeDtypeStruct((NP, PW), i32),
    ],
)


def _fin_body(sa_ref, sb_ref, ya_ref, yb_ref, dis_ref, b_ref, batch_ref,
              wfc_ref, bfc_ref, out_ref, acc, cnt):
    i = pl.program_id(0)

    @pl.when(i == 0)
    def _():
        acc[...] = jnp.zeros_like(acc)
        cnt[...] = jnp.zeros_like(cnt)

    dis = dis_ref[...]
    ha = (sa_ref[...] + _unpack_half(ya_ref)) * dis
    hb = (sb_ref[...] + _unpack_half(yb_ref)) * dis
    h = jnp.maximum(jnp.concatenate([ha, hb], axis=1) + b_ref[...], 0.0)
    gid = lax.broadcasted_iota(i32, (G, BR), 0)
    onehot = (gid == batch_ref[...]).astype(f32)
    acc[...] += jnp.dot(onehot, h, preferred_element_type=f32)
    cnt[...] += jnp.broadcast_to(
        jnp.sum(onehot, axis=1, keepdims=True), (G, D))

    @pl.when(i == NP // BR - 1)
    def _():
        g = acc[...] / jnp.maximum(cnt[...], 1.0)
        out_ref[...] = (jnp.dot(g, wfc_ref[...], preferred_element_type=f32)
                        + bfc_ref[...])


_fin = pl.pallas_call(
    _fin_body,
    grid=(NP // BR,),
    in_specs=[
        pl.BlockSpec((BR, HALF), lambda i: (i, 0)),
        pl.BlockSpec((BR, HALF), lambda i: (i, 0)),
        pl.BlockSpec((BR, PW), lambda i: (i, 0)),
        pl.BlockSpec((BR, PW), lambda i: (i, 0)),
        pl.BlockSpec((BR, HALF), lambda i: (i, 0)),
        pl.BlockSpec((1, D), lambda i: (0, 0)),
        pl.BlockSpec((1, BR), lambda i: (0, i)),
        pl.BlockSpec((D, HALF), lambda i: (0, 0)),
        pl.BlockSpec((1, HALF), lambda i: (0, 0)),
    ],
    out_specs=pl.BlockSpec((G, HALF), lambda i: (0, 0)),
    out_shape=jax.ShapeDtypeStruct((G, HALF), f32),
    scratch_shapes=[pltpu.VMEM((G, D), f32), pltpu.VMEM((G, D), f32)],
)


@jax.jit
def kernel(x, edge_index, batch, W1, b1, W2, b2, Wfc, bfc):
    xp = jnp.zeros((NP, D), f32).at[:N, :].set(x.astype(f32))
    src = edge_index[0].astype(i32)
    dst = edge_index[1].astype(i32)
    srcp = jnp.full((EP,), NP - 1, i32).at[:E].set(src)
    dstp = jnp.full((EP,), NP - 1, i32).at[:E].set(dst)
    dst32 = dstp.reshape(32, NCH_H, 128)
    src16 = srcp.reshape(NTILES, NCH_A, CH)
    dst16 = dstp.reshape(NTILES, NCH_A, CH)
    ones_h = jnp.ones((128, 16), f32)
    zeros16 = jnp.zeros((NP, 16), f32)
    zeros_h = jnp.zeros((NP, HALF), f32)
    batch_row = jnp.full((1, NP), G, i32).at[0, :N].set(batch.astype(i32))
    b1r = b1.reshape(1, D).astype(f32)
    b2r = b2.reshape(1, D).astype(f32)
    wfc_p = jnp.zeros((D, HALF), f32).at[:, :C].set(Wfc.astype(f32))
    bfc_p = jnp.zeros((1, HALF), f32).at[0, :C].set(bfc.astype(f32))

    hist_k, agg_k = _sc_kernels()
    hist = hist_k(dst32, ones_h, zeros16)
    ya, yb, dis = _mm1(xp, W1.astype(f32), hist)
    s1a, s1b = agg_k(ya, yb, src16, dst16, zeros_h)
    y2a, y2b = _mm2(s1a, s1b, ya, yb, dis, b1r, W2.astype(f32))
    s2a, s2b = agg_k(y2a, y2b, src16, dst16, zeros_h)
    outp = _fin(s2a, s2b, y2a, y2b, dis, b2r, batch_row, wfc_p, bfc_p)
    return outp[:, :C]
